# Initial kernel scaffold; baseline (speedup 1.0000x reference)
import jax
import jax.numpy as jnp
from jax.experimental import pallas as pl


def kernel(labels, class_means, class_stds, noise):
    raise NotImplementedError("write your pallas kernel here")


# speedup vs baseline: 1.7271x; 1.7271x over previous
import functools

import jax
import jax.numpy as jnp
from jax import lax
from jax.experimental import pallas as pl
from jax.experimental.pallas import tpu as pltpu
from jax.experimental.pallas import tpu_sc as plsc

NC = 2
NS = 16
L = 16
NW = NC * NS


def _embed(B, D, labels, means, stds, noise):
    BPW = B // NW
    CH = 8
    NCHUNK = BPW // CH
    PER_ROW = D // L

    mesh = plsc.VectorSubcoreMesh(core_axis_name="c", subcore_axis_name="s")

    @functools.partial(
        pl.kernel,
        mesh=mesh,
        out_type=jax.ShapeDtypeStruct((B, D), jnp.float32),
        scratch_types=[
            pltpu.VMEM((CH,), jnp.int32),
            pltpu.VMEM((CH, D), jnp.float32),
            pltpu.VMEM((CH, D), jnp.float32),
            pltpu.VMEM((CH, D), jnp.float32),
            pltpu.SemaphoreType.DMA,
            pltpu.SemaphoreType.DMA,
            pltpu.SemaphoreType.DMA,
        ],
    )
    def k(labels_hbm, means_hbm, stds_hbm, noise_hbm, out_hbm,
          idx_v, mean_v, std_v, noise_v, sem_m, sem_s, sem_n):
        wid = lax.axis_index("s") * NC + lax.axis_index("c")
        base = wid * BPW

        for c in range(NCHUNK):
            row0 = base + c * CH
            pltpu.sync_copy(labels_hbm.at[pl.ds(row0, CH)], idx_v)
            cm = pltpu.async_copy(means_hbm.at[idx_v], mean_v, sem_m)
            cs = pltpu.async_copy(stds_hbm.at[idx_v], std_v, sem_s)
            cn = pltpu.async_copy(noise_hbm.at[pl.ds(row0, CH)],
                                  noise_v, sem_n)
            cm.wait()
            cs.wait()
            cn.wait()
            for r in range(CH):
                def step(j, _, r=r):
                    sl = pl.ds(j * L, L)
                    mean_v[r, sl] = (mean_v[r, sl]
                                     + std_v[r, sl] * noise_v[r, sl])
                    return 0
                lax.fori_loop(0, PER_ROW, step, 0)
            pltpu.sync_copy(mean_v, out_hbm.at[pl.ds(row0, CH)])

    return k(labels, means, stds, noise)


def kernel(labels, class_means, class_stds, noise):
    num_classes = class_means.shape[0]
    B = labels.shape[0]
    D = class_means.size // num_classes
    out = _embed(
        B, D,
        labels.astype(jnp.int32),
        class_means.reshape(num_classes, D),
        class_stds.reshape(num_classes, D),
        noise.reshape(B, D),
    )
    return out.reshape(noise.shape)


# speedup vs baseline: 2.2311x; 1.2918x over previous
import functools

import jax
import jax.numpy as jnp
from jax import lax
from jax.experimental import pallas as pl
from jax.experimental.pallas import tpu as pltpu
from jax.experimental.pallas import tpu_sc as plsc

NC = 2
NS = 16
L = 16
NW = NC * NS


def _embed(B, D, labels, means, stds, noise):
    BPW = B // NW
    CH = 8
    NCHUNK = BPW // CH
    PER_ROW = D // L

    mesh = plsc.VectorSubcoreMesh(core_axis_name="c", subcore_axis_name="s")

    @functools.partial(
        pl.kernel,
        mesh=mesh,
        out_type=jax.ShapeDtypeStruct((B, D), jnp.float32),
        scratch_types=[
            pltpu.VMEM((CH,), jnp.int32),
            pltpu.VMEM((CH, D), jnp.float32),
            pltpu.VMEM((CH, D), jnp.float32),
            pltpu.VMEM((CH, D), jnp.float32),
            pltpu.SemaphoreType.DMA,
            pltpu.SemaphoreType.DMA,
            pltpu.SemaphoreType.DMA,
        ],
    )
    def k(labels_hbm, means_hbm, stds_hbm, noise_hbm, out_hbm,
          idx_v, mean_v, std_v, noise_v, sem_m, sem_s, sem_n):
        wid = lax.axis_index("s") * NC + lax.axis_index("c")
        base = wid * BPW

        @pl.loop(0, NCHUNK)
        def _(c):
            row0 = base + c * CH
            pltpu.sync_copy(labels_hbm.at[pl.ds(row0, CH)], idx_v)
            cm = pltpu.async_copy(means_hbm.at[idx_v], mean_v, sem_m)
            cs = pltpu.async_copy(stds_hbm.at[idx_v], std_v, sem_s)
            cn = pltpu.async_copy(noise_hbm.at[pl.ds(row0, CH)],
                                  noise_v, sem_n)
            cm.wait()
            cs.wait()
            cn.wait()
            for r in range(CH):
                @plsc.parallel_loop(0, D, step=L, unroll=8)
                def _(j, r=r):
                    sl = pl.ds(j, L)
                    mean_v[r, sl] = (mean_v[r, sl]
                                     + std_v[r, sl] * noise_v[r, sl])
            pltpu.sync_copy(mean_v, out_hbm.at[pl.ds(row0, CH)])

    return k(labels, means, stds, noise)


def kernel(labels, class_means, class_stds, noise):
    num_classes = class_means.shape[0]
    B = labels.shape[0]
    D = class_means.size // num_classes
    out = _embed(
        B, D,
        labels.astype(jnp.int32),
        class_means.reshape(num_classes, D),
        class_stds.reshape(num_classes, D),
        noise.reshape(B, D),
    )
    return out.reshape(noise.shape)


# speedup vs baseline: 2.8059x; 1.2576x over previous
import functools

import jax
import jax.numpy as jnp
from jax import lax
from jax.experimental import pallas as pl
from jax.experimental.pallas import tpu as pltpu
from jax.experimental.pallas import tpu_sc as plsc

NC = 2
NS = 16
L = 16
NW = NC * NS
CH = 2
NBUF = 4


def _embed(B, D, labels2, means, stds, noise):
    BPW = B // NW
    NCHUNK = BPW // CH

    mesh = plsc.VectorSubcoreMesh(core_axis_name="c", subcore_axis_name="s")

    @functools.partial(
        pl.kernel,
        mesh=mesh,
        out_type=jax.ShapeDtypeStruct((B, D), jnp.float32),
        scratch_types=(
            [pltpu.VMEM((NCHUNK, CH), jnp.int32),
             pltpu.VMEM((NBUF, CH, D), jnp.float32),
             pltpu.VMEM((NBUF, CH, D), jnp.float32),
             pltpu.VMEM((NBUF, CH, D), jnp.float32)]
            + [pltpu.SemaphoreType.DMA] * (4 * NBUF)
        ),
    )
    def k(labels_hbm, means_hbm, stds_hbm, noise_hbm, out_hbm,
          idx_v, mbuf, sbuf, nbuf, *sems):
        sem_m = sems[0:NBUF]
        sem_s = sems[NBUF:2 * NBUF]
        sem_n = sems[2 * NBUF:3 * NBUF]
        sem_o = sems[3 * NBUF:4 * NBUF]

        wid = lax.axis_index("s") * NC + lax.axis_index("c")
        base = wid * BPW

        pltpu.sync_copy(labels_hbm.at[pl.ds(wid * NCHUNK, NCHUNK)], idx_v)

        def start_inputs(c, b):
            pltpu.async_copy(means_hbm.at[idx_v.at[c]], mbuf.at[b], sem_m[b])
            pltpu.async_copy(stds_hbm.at[idx_v.at[c]], sbuf.at[b], sem_s[b])
            pltpu.async_copy(noise_hbm.at[pl.ds(base + c * CH, CH)],
                             nbuf.at[b], sem_n[b])

        def slot(c, b, first, prefetch):
            pltpu.make_async_copy(
                means_hbm.at[idx_v.at[c]], mbuf.at[b], sem_m[b]).wait()
            pltpu.make_async_copy(
                stds_hbm.at[idx_v.at[c]], sbuf.at[b], sem_s[b]).wait()
            pltpu.make_async_copy(
                noise_hbm.at[pl.ds(base + c * CH, CH)], nbuf.at[b],
                sem_n[b]).wait()
            for r in range(CH):
                @plsc.parallel_loop(0, D, step=L, unroll=8)
                def _(j, r=r, b=b):
                    sl = pl.ds(j, L)
                    mbuf[b, r, sl] = (mbuf[b, r, sl]
                                      + sbuf[b, r, sl] * nbuf[b, r, sl])
            pltpu.async_copy(
                mbuf.at[b], out_hbm.at[pl.ds(base + c * CH, CH)], sem_o[b])
            if prefetch:
                cn = c + (NBUF - 1)
                p = (b + NBUF - 1) % NBUF
                if not first:
                    pltpu.make_async_copy(
                        mbuf.at[p],
                        out_hbm.at[pl.ds(base + (c - 1) * CH, CH)],
                        sem_o[p]).wait()
                start_inputs(cn, p)

        for b in range(NBUF - 1):
            start_inputs(b, b)

        slot(0, 0, first=True, prefetch=True)

        @pl.loop(0, (NCHUNK - NBUF) // NBUF)
        def _(g):
            c0 = 1 + g * NBUF
            for i in range(NBUF):
                slot(c0 + i, (1 + i) % NBUF, first=False, prefetch=True)

        for i in range(NBUF - 1):
            c = NCHUNK - (NBUF - 1) + i
            slot(c, c % NBUF, first=False, prefetch=False)

        for i in range(NBUF):
            c = NCHUNK - NBUF + i
            b = c % NBUF
            pltpu.make_async_copy(
                mbuf.at[b], out_hbm.at[pl.ds(base + c * CH, CH)],
                sem_o[b]).wait()

    return k(labels2, means, stds, noise)


def kernel(labels, class_means, class_stds, noise):
    num_classes = class_means.shape[0]
    B = labels.shape[0]
    D = class_means.size // num_classes
    out = _embed(
        B, D,
        labels.astype(jnp.int32).reshape(B // CH, CH),
        class_means.reshape(num_classes, D),
        class_stds.reshape(num_classes, D),
        noise.reshape(B, D),
    )
    return out.reshape(noise.shape)
